# Initial kernel scaffold; baseline (speedup 1.0000x reference)
import jax
import jax.numpy as jnp
from jax.experimental import pallas as pl


def kernel(hidden_states, gate_w, w_gate, w_up, w_down, ws_gate, ws_up, ws_down):
    raise NotImplementedError("write your pallas kernel here")


# speedup vs baseline: 1.4572x; 1.4572x over previous
import jax
import jax.numpy as jnp
from jax.experimental import pallas as pl


def _router_kernel(x_ref, gw_ref, cmb_ref):
    x = x_ref[...].astype(jnp.bfloat16)
    gw = gw_ref[...].astype(jnp.bfloat16)
    logits = jax.lax.dot_general(
        x, gw, (((1,), (1,)), ((), ())),
        preferred_element_type=jnp.float32,
    )
    t, e = logits.shape
    cols = jax.lax.broadcasted_iota(jnp.int32, (t, e), 1)
    m1 = jnp.max(logits, axis=1, keepdims=True)
    i1 = jnp.min(jnp.where(logits == m1, cols, e), axis=1, keepdims=True)
    mask1 = cols == i1
    l2 = jnp.where(mask1, -jnp.inf, logits)
    m2 = jnp.max(l2, axis=1, keepdims=True)
    i2 = jnp.min(jnp.where(l2 == m2, cols, e), axis=1, keepdims=True)
    mask2 = cols == i2
    p1 = 1.0 / (1.0 + jnp.exp(m2 - m1))
    cmb_ref[...] = jnp.where(mask1, p1, 0.0) + jnp.where(mask2, 1.0 - p1, 0.0)


def _moe_kernel(TB, xb_ref, cmb_ref, wg_ref, wu_ref, wd_ref,
                wsg_ref, wsu_ref, wsd_ref, out_ref):
    e = pl.program_id(0)
    t = pl.program_id(1)
    rows = pl.ds(t * TB, TB)
    xb = xb_ref[rows, :]

    g = jax.lax.dot_general(xb, wg_ref[0], (((1,), (1,)), ((), ())),
                            preferred_element_type=jnp.float32)
    u = jax.lax.dot_general(xb, wu_ref[0], (((1,), (1,)), ((), ())),
                            preferred_element_type=jnp.float32)
    h = (g * jax.nn.sigmoid(g) * u).astype(jnp.bfloat16)
    o = jax.lax.dot_general(h, wd_ref[0], (((1,), (1,)), ((), ())),
                            preferred_element_type=jnp.float32)

    cmb = cmb_ref[rows, :]
    n_e = cmb.shape[1]
    cols = jax.lax.broadcasted_iota(jnp.int32, cmb.shape, 1)
    wcol = jnp.sum(jnp.where(cols == e, cmb, 0.0), axis=1, keepdims=True)
    contrib = o * wcol

    @pl.when(e == 0)
    def _init():
        gs = jax.lax.dot_general(xb, wsg_ref[...], (((1,), (1,)), ((), ())),
                                 preferred_element_type=jnp.float32)
        us = jax.lax.dot_general(xb, wsu_ref[...], (((1,), (1,)), ((), ())),
                                 preferred_element_type=jnp.float32)
        hs = (gs * jax.nn.sigmoid(gs) * us).astype(jnp.bfloat16)
        sh = jax.lax.dot_general(hs, wsd_ref[...], (((1,), (1,)), ((), ())),
                                 preferred_element_type=jnp.float32)
        out_ref[rows, :] = contrib + sh

    @pl.when(e != 0)
    def _accum():
        out_ref[rows, :] = out_ref[rows, :] + contrib

    del n_e


def kernel(hidden_states, gate_w, w_gate, w_up, w_down, ws_gate, ws_up, ws_down):
    orig_shape = hidden_states.shape
    x = hidden_states.reshape(-1, orig_shape[-1])
    T, D = x.shape
    E, FF, _ = w_gate.shape
    SFF = ws_gate.shape[0]
    TB = 512
    NTB = T // TB

    combine = pl.pallas_call(
        _router_kernel,
        out_shape=jax.ShapeDtypeStruct((T, E), jnp.float32),
    )(x, gate_w)

    xb = x.astype(jnp.bfloat16)
    wg = w_gate.astype(jnp.bfloat16)
    wu = w_up.astype(jnp.bfloat16)
    wd = w_down.astype(jnp.bfloat16)
    wsg = ws_gate.astype(jnp.bfloat16)
    wsu = ws_up.astype(jnp.bfloat16)
    wsd = ws_down.astype(jnp.bfloat16)

    import functools
    y = pl.pallas_call(
        functools.partial(_moe_kernel, TB),
        grid=(E, NTB),
        in_specs=[
            pl.BlockSpec((T, D), lambda e, t: (0, 0)),
            pl.BlockSpec((T, E), lambda e, t: (0, 0)),
            pl.BlockSpec((1, FF, D), lambda e, t: (e, 0, 0)),
            pl.BlockSpec((1, FF, D), lambda e, t: (e, 0, 0)),
            pl.BlockSpec((1, D, FF), lambda e, t: (e, 0, 0)),
            pl.BlockSpec((SFF, D), lambda e, t: (0, 0)),
            pl.BlockSpec((SFF, D), lambda e, t: (0, 0)),
            pl.BlockSpec((D, SFF), lambda e, t: (0, 0)),
        ],
        out_specs=pl.BlockSpec((T, D), lambda e, t: (0, 0)),
        out_shape=jax.ShapeDtypeStruct((T, D), jnp.float32),
    )(xb, combine, wg, wu, wd, wsg, wsu, wsd)

    return y.reshape(orig_shape)
